# Initial kernel scaffold; baseline (speedup 1.0000x reference)
import jax
import jax.numpy as jnp
from jax.experimental import pallas as pl


def kernel(loc_data, conf_data, landm_data, priors, targets):
    raise NotImplementedError("write your pallas kernel here")


# speedup vs baseline: 46.9619x; 46.9619x over previous
import jax
import jax.numpy as jnp
from jax.experimental import pallas as pl
from jax.experimental.pallas import tpu as pltpu

_THRESHOLD = 0.35
_VAR0, _VAR1 = 0.1, 0.2
_NEGPOS = 7
_MAXFLOAT_BITS = 0x7F800000


def _body(loc_ref, conf_ref, landm_ref, priors_ref, tgt_ref, out_ref):
    nobj = tgt_ref.shape[1]
    num_p = loc_ref.shape[2]
    f32 = jnp.float32

    pr = priors_ref[...]
    pcx, pcy = pr[0:1, :], pr[1:2, :]
    pw, ph = pr[2:3, :], pr[3:4, :]
    px0 = pcx - pw * 0.5
    py0 = pcy - ph * 0.5
    px1 = pcx + pw * 0.5
    py1 = pcy + ph * 0.5
    area_p = (px1 - px0) * (py1 - py0)

    tgt = tgt_ref[0]
    tx0, ty0 = tgt[:, 0:1], tgt[:, 1:2]
    tx1, ty1 = tgt[:, 2:3], tgt[:, 3:4]
    area_t = (tx1 - tx0) * (ty1 - ty0)

    iw = jnp.maximum(jnp.minimum(tx1, px1) - jnp.maximum(tx0, px0), 0.0)
    ih = jnp.maximum(jnp.minimum(ty1, py1) - jnp.maximum(ty0, py0), 0.0)
    inter = iw * ih
    ov = inter / (area_t + area_p - inter)

    ji = jax.lax.broadcasted_iota(jnp.int32, (nobj, num_p), 0)
    pi = jax.lax.broadcasted_iota(jnp.int32, (nobj, num_p), 1)

    bto = jnp.max(ov, axis=0, keepdims=True)
    bti = jnp.min(jnp.where(ov == bto, ji, nobj), axis=0, keepdims=True)
    bpo = jnp.max(ov, axis=1, keepdims=True)
    bpi = jnp.min(jnp.where(ov == bpo, pi, num_p), axis=1, keepdims=True)
    valid = bpo >= 0.2
    has_valid = jnp.any(valid)

    eq = bpi == pi
    assigned = jnp.max(jnp.where(eq, ji, -1), axis=0, keepdims=True)
    forced = jnp.any(eq & valid, axis=0, keepdims=True)
    bti = jnp.where(assigned >= 0, assigned, bti)
    bto = jnp.where(forced, 2.0, bto)
    pos = (bto >= _THRESHOLD) & has_valid
    posf = pos.astype(f32)

    onehot = (bti == ji).astype(f32)
    tl = jnp.transpose(tgt[:, 0:14])
    matched = jax.lax.dot_general(tl, onehot, (((1,), (0,)), ((), ())),
                                  preferred_element_type=f32)

    m0, m1 = matched[0:1], matched[1:2]
    m2, m3 = matched[2:3], matched[3:4]
    g_cx = ((m0 + m2) * 0.5 - pcx) / (_VAR0 * pw)
    g_cy = ((m1 + m3) * 0.5 - pcy) / (_VAR0 * ph)
    g_w = jnp.log((m2 - m0) / pw) / _VAR1
    g_h = jnp.log((m3 - m1) / ph) / _VAR1
    loc_t = jnp.concatenate([g_cx, g_cy, g_w, g_h], axis=0)

    lm_c = jnp.concatenate([pcx, pcy] * 5, axis=0)
    lm_s = jnp.concatenate([pw, ph] * 5, axis=0) * _VAR0
    landm_t = (matched[4:14] - lm_c) / lm_s

    d = loc_ref[0] - loc_t
    ad = jnp.abs(d)
    loss_l = jnp.sum(jnp.where(ad < 1.0, 0.5 * d * d, ad - 0.5) * posf)
    d2 = landm_ref[0] - landm_t
    ad2 = jnp.abs(d2)
    loss_lm = jnp.sum(jnp.where(ad2 < 1.0, 0.5 * d2 * d2, ad2 - 0.5) * posf)

    cf = conf_ref[0]
    x0, x1 = cf[0:1, :], cf[1:2, :]
    mx = jnp.maximum(x0, x1)
    lse = jnp.log(jnp.exp(x0 - mx) + jnp.exp(x1 - mx)) + mx
    ce0 = lse - x0
    ce1 = lse - x1
    mine = jnp.where(pos, 0.0, ce0)

    npos = jnp.sum(posf)
    k = jnp.minimum(_NEGPOS * npos.astype(jnp.int32), num_p - 1)

    bits = jax.lax.bitcast_convert_type(mine, jnp.int32)

    def _count_gt(t):
        return jnp.sum((bits > t).astype(jnp.int32))

    def _step(_, lohi):
        lo, hi = lohi
        mid = (lo + hi) // 2
        ge = _count_gt(mid) >= k
        return jnp.where(ge, mid, lo), jnp.where(ge, hi, mid)

    _, thr = jax.lax.fori_loop(
        0, 31, _step, (jnp.int32(-1), jnp.int32(_MAXFLOAT_BITS)))
    n_gt = _count_gt(thr)
    rem = k - n_gt
    eqm = bits == thr
    eqi = eqm.astype(jnp.int32)
    cum = eqi
    shift = 1
    while shift < num_p:
        cum = cum + jnp.concatenate(
            [jnp.zeros((1, shift), jnp.int32), cum[:, :num_p - shift]], axis=1)
        shift *= 2
    cum_excl = cum - eqi
    sel_neg = (bits > thr) | (eqm & (cum_excl < rem))
    sel_neg = sel_neg & (k > 0)

    loss_c = jnp.sum(jnp.where(pos, ce1, jnp.where(sel_neg, ce0, 0.0)))

    zeros = jnp.zeros((1, 124), f32)
    row = jnp.concatenate(
        [loss_l.reshape(1, 1), loss_c.reshape(1, 1),
         loss_lm.reshape(1, 1), npos.reshape(1, 1), zeros], axis=1)
    out_ref[0, :, :] = row


def kernel(loc_data, conf_data, landm_data, priors, targets):
    b, p = loc_data.shape[0], loc_data.shape[1]
    nobj = targets.shape[1]
    loc_cm = jnp.swapaxes(loc_data, 1, 2)
    conf_cm = jnp.swapaxes(conf_data, 1, 2)
    landm_cm = jnp.swapaxes(landm_data, 1, 2)
    pri_cm = jnp.transpose(priors)

    parts = pl.pallas_call(
        _body,
        grid=(b,),
        in_specs=[
            pl.BlockSpec((1, 4, p), lambda i: (i, 0, 0)),
            pl.BlockSpec((1, 2, p), lambda i: (i, 0, 0)),
            pl.BlockSpec((1, 10, p), lambda i: (i, 0, 0)),
            pl.BlockSpec((4, p), lambda i: (0, 0)),
            pl.BlockSpec((1, nobj, 15), lambda i: (i, 0, 0)),
        ],
        out_specs=pl.BlockSpec((1, 1, 128), lambda i: (i, 0, 0)),
        out_shape=jax.ShapeDtypeStruct((b, 1, 128), jnp.float32),
        compiler_params=pltpu.CompilerParams(
            dimension_semantics=("parallel",)),
    )(loc_cm, conf_cm, landm_cm, pri_cm, targets)

    s = jnp.sum(parts[:, 0, :4], axis=0)
    n = jnp.maximum(s[3], 1.0)
    return jnp.stack([s[0] / n, s[1] / n, s[2] / n])


# speedup vs baseline: 47.0206x; 1.0013x over previous
import jax
import jax.numpy as jnp
from jax.experimental import pallas as pl
from jax.experimental.pallas import tpu as pltpu

_THRESHOLD = 0.35
_VAR0, _VAR1 = 0.1, 0.2
_NEGPOS = 7
_MAXFLOAT_BITS = 0x7F800000


def _body(loc_ref, conf_ref, landm_ref, priors_ref, tgt_ref, out_ref):
    nobj = tgt_ref.shape[1]
    num_p = loc_ref.shape[2]
    f32 = jnp.float32

    pr = priors_ref[...]
    pcx, pcy = pr[0:1, :], pr[1:2, :]
    pw, ph = pr[2:3, :], pr[3:4, :]
    px0 = pcx - pw * 0.5
    py0 = pcy - ph * 0.5
    px1 = pcx + pw * 0.5
    py1 = pcy + ph * 0.5
    area_p = (px1 - px0) * (py1 - py0)

    tgt = tgt_ref[0]
    tx0, ty0 = tgt[:, 0:1], tgt[:, 1:2]
    tx1, ty1 = tgt[:, 2:3], tgt[:, 3:4]
    area_t = (tx1 - tx0) * (ty1 - ty0)

    iw = jnp.maximum(jnp.minimum(tx1, px1) - jnp.maximum(tx0, px0), 0.0)
    ih = jnp.maximum(jnp.minimum(ty1, py1) - jnp.maximum(ty0, py0), 0.0)
    inter = iw * ih
    ov = inter / (area_t + area_p - inter)

    ji = jax.lax.broadcasted_iota(jnp.int32, (nobj, num_p), 0)
    pi = jax.lax.broadcasted_iota(jnp.int32, (nobj, num_p), 1)

    bto = jnp.max(ov, axis=0, keepdims=True)
    bti = jnp.min(jnp.where(ov == bto, ji, nobj), axis=0, keepdims=True)
    bpo = jnp.max(ov, axis=1, keepdims=True)
    bpi = jnp.min(jnp.where(ov == bpo, pi, num_p), axis=1, keepdims=True)
    valid = bpo >= 0.2
    has_valid = jnp.any(valid)

    eq = bpi == pi
    assigned = jnp.max(jnp.where(eq, ji, -1), axis=0, keepdims=True)
    forced = jnp.any(eq & valid, axis=0, keepdims=True)
    bti = jnp.where(assigned >= 0, assigned, bti)
    bto = jnp.where(forced, 2.0, bto)
    pos = (bto >= _THRESHOLD) & has_valid
    posf = pos.astype(f32)

    onehot = (bti == ji).astype(f32)
    tl = jnp.transpose(tgt[:, 0:14])
    matched = jax.lax.dot_general(tl, onehot, (((1,), (0,)), ((), ())),
                                  preferred_element_type=f32)

    m0, m1 = matched[0:1], matched[1:2]
    m2, m3 = matched[2:3], matched[3:4]
    g_cx = ((m0 + m2) * 0.5 - pcx) / (_VAR0 * pw)
    g_cy = ((m1 + m3) * 0.5 - pcy) / (_VAR0 * ph)
    g_w = jnp.log((m2 - m0) / pw) / _VAR1
    g_h = jnp.log((m3 - m1) / ph) / _VAR1
    loc_t = jnp.concatenate([g_cx, g_cy, g_w, g_h], axis=0)

    lm_c = jnp.concatenate([pcx, pcy] * 5, axis=0)
    lm_s = jnp.concatenate([pw, ph] * 5, axis=0) * _VAR0
    landm_t = (matched[4:14] - lm_c) / lm_s

    d = loc_ref[0] - loc_t
    ad = jnp.abs(d)
    loss_l = jnp.sum(jnp.where(ad < 1.0, 0.5 * d * d, ad - 0.5) * posf)
    d2 = landm_ref[0] - landm_t
    ad2 = jnp.abs(d2)
    loss_lm = jnp.sum(jnp.where(ad2 < 1.0, 0.5 * d2 * d2, ad2 - 0.5) * posf)

    cf = conf_ref[0]
    x0, x1 = cf[0:1, :], cf[1:2, :]
    mx = jnp.maximum(x0, x1)
    lse = jnp.log(jnp.exp(x0 - mx) + jnp.exp(x1 - mx)) + mx
    ce0 = lse - x0
    ce1 = lse - x1
    mine = jnp.where(pos, 0.0, ce0)

    npos = jnp.sum(posf)
    k = jnp.minimum(_NEGPOS * npos.astype(jnp.int32), num_p - 1)

    bits = jax.lax.bitcast_convert_type(mine, jnp.int32)

    def _count_gt(t):
        return jnp.sum((bits > t).astype(jnp.int32))

    def _step(_, lohi):
        lo, hi = lohi
        mid = (lo + hi) // 2
        ge = _count_gt(mid) >= k
        return jnp.where(ge, mid, lo), jnp.where(ge, hi, mid)

    _, thr = jax.lax.fori_loop(
        0, 31, _step, (jnp.int32(-1), jnp.int32(_MAXFLOAT_BITS)))
    n_gt = _count_gt(thr)
    rem = k - n_gt
    eqm = bits == thr
    eqi = eqm.astype(jnp.int32)
    cum = eqi
    shift = 1
    while shift < num_p:
        cum = cum + jnp.concatenate(
            [jnp.zeros((1, shift), jnp.int32), cum[:, :num_p - shift]],
            axis=1)
        shift *= 2
    sel_eq = eqm & ((cum - eqi) < rem)
    sel_neg = ((bits > thr) | sel_eq) & (k > 0)

    loss_c = jnp.sum(jnp.where(pos, ce1, jnp.where(sel_neg, ce0, 0.0)))

    zeros = jnp.zeros((1, 124), f32)
    row = jnp.concatenate(
        [loss_l.reshape(1, 1), loss_c.reshape(1, 1),
         loss_lm.reshape(1, 1), npos.reshape(1, 1), zeros], axis=1)
    out_ref[0, :, :] = row


def kernel(loc_data, conf_data, landm_data, priors, targets):
    b, p = loc_data.shape[0], loc_data.shape[1]
    nobj = targets.shape[1]
    loc_cm = jnp.swapaxes(loc_data, 1, 2)
    conf_cm = jnp.swapaxes(conf_data, 1, 2)
    landm_cm = jnp.swapaxes(landm_data, 1, 2)
    pri_cm = jnp.transpose(priors)

    parts = pl.pallas_call(
        _body,
        grid=(b,),
        in_specs=[
            pl.BlockSpec((1, 4, p), lambda i: (i, 0, 0)),
            pl.BlockSpec((1, 2, p), lambda i: (i, 0, 0)),
            pl.BlockSpec((1, 10, p), lambda i: (i, 0, 0)),
            pl.BlockSpec((4, p), lambda i: (0, 0)),
            pl.BlockSpec((1, nobj, 15), lambda i: (i, 0, 0)),
        ],
        out_specs=pl.BlockSpec((1, 1, 128), lambda i: (i, 0, 0)),
        out_shape=jax.ShapeDtypeStruct((b, 1, 128), jnp.float32),
        compiler_params=pltpu.CompilerParams(
            dimension_semantics=("parallel",)),
    )(loc_cm, conf_cm, landm_cm, pri_cm, targets)

    s = jnp.sum(parts[:, 0, :4], axis=0)
    n = jnp.maximum(s[3], 1.0)
    return jnp.stack([s[0] / n, s[1] / n, s[2] / n])


# speedup vs baseline: 85.0620x; 1.8090x over previous
import jax
import jax.numpy as jnp
from jax.experimental import pallas as pl
from jax.experimental.pallas import tpu as pltpu

_THRESHOLD = 0.35
_VAR0, _VAR1 = 0.1, 0.2
_NEGPOS = 7
_MAXFLOAT_BITS = 0x7F800000


def _body(loc_ref, conf_ref, landm_ref, priors_ref, tgt_ref, out_ref):
    img = loc_ref.shape[0]
    nobj = tgt_ref.shape[1]
    num_p = loc_ref.shape[2]
    f32 = jnp.float32

    pr = priors_ref[...]
    pcx, pcy = pr[0:1, :], pr[1:2, :]
    pw, ph = pr[2:3, :], pr[3:4, :]
    px0 = pcx - pw * 0.5
    py0 = pcy - ph * 0.5
    px1 = pcx + pw * 0.5
    py1 = pcy + ph * 0.5
    area_p = (px1 - px0) * (py1 - py0)

    ji = jax.lax.broadcasted_iota(jnp.int32, (nobj, num_p), 0)
    pi = jax.lax.broadcasted_iota(jnp.int32, (nobj, num_p), 1)

    lm_c = jnp.concatenate([pcx, pcy] * 5, axis=0)
    lm_s = jnp.concatenate([pw, ph] * 5, axis=0) * _VAR0

    mine_rows, ce0_rows, ce1_rows, pos_rows = [], [], [], []
    ll_list, lm_list, np_list = [], [], []

    for b in range(img):
        tgt = tgt_ref[b]
        tx0, ty0 = tgt[:, 0:1], tgt[:, 1:2]
        tx1, ty1 = tgt[:, 2:3], tgt[:, 3:4]
        area_t = (tx1 - tx0) * (ty1 - ty0)

        iw = jnp.maximum(jnp.minimum(tx1, px1) - jnp.maximum(tx0, px0), 0.0)
        ih = jnp.maximum(jnp.minimum(ty1, py1) - jnp.maximum(ty0, py0), 0.0)
        inter = iw * ih
        ov = inter / (area_t + area_p - inter)

        bto = jnp.max(ov, axis=0, keepdims=True)
        bti = jnp.min(jnp.where(ov == bto, ji, nobj), axis=0, keepdims=True)
        bpo = jnp.max(ov, axis=1, keepdims=True)
        bpi = jnp.min(jnp.where(ov == bpo, pi, num_p), axis=1, keepdims=True)
        valid = bpo >= 0.2
        has_valid = jnp.any(valid)

        eq = bpi == pi
        assigned = jnp.max(jnp.where(eq, ji, -1), axis=0, keepdims=True)
        forced = jnp.any(eq & valid, axis=0, keepdims=True)
        bti = jnp.where(assigned >= 0, assigned, bti)
        bto = jnp.where(forced, 2.0, bto)
        pos = (bto >= _THRESHOLD) & has_valid
        posf = pos.astype(f32)

        onehot = (bti == ji).astype(f32)
        tl = jnp.transpose(tgt[:, 0:14])
        matched = jax.lax.dot_general(tl, onehot, (((1,), (0,)), ((), ())),
                                      preferred_element_type=f32)

        m0, m1 = matched[0:1], matched[1:2]
        m2, m3 = matched[2:3], matched[3:4]
        g_cx = ((m0 + m2) * 0.5 - pcx) / (_VAR0 * pw)
        g_cy = ((m1 + m3) * 0.5 - pcy) / (_VAR0 * ph)
        g_w = jnp.log((m2 - m0) / pw) / _VAR1
        g_h = jnp.log((m3 - m1) / ph) / _VAR1
        loc_t = jnp.concatenate([g_cx, g_cy, g_w, g_h], axis=0)
        landm_t = (matched[4:14] - lm_c) / lm_s

        d = loc_ref[b] - loc_t
        ad = jnp.abs(d)
        ll = jnp.sum(jnp.where(ad < 1.0, 0.5 * d * d, ad - 0.5) * posf)
        d2 = landm_ref[b] - landm_t
        ad2 = jnp.abs(d2)
        lm = jnp.sum(jnp.where(ad2 < 1.0, 0.5 * d2 * d2, ad2 - 0.5) * posf)

        cf = conf_ref[b]
        x0, x1 = cf[0:1, :], cf[1:2, :]
        mx = jnp.maximum(x0, x1)
        lse = jnp.log(jnp.exp(x0 - mx) + jnp.exp(x1 - mx)) + mx
        ce0 = lse - x0
        ce1 = lse - x1
        mine_rows.append(jnp.where(pos, 0.0, ce0))
        ce0_rows.append(ce0)
        ce1_rows.append(ce1)
        pos_rows.append(posf)
        ll_list.append(ll.reshape(1, 1))
        lm_list.append(lm.reshape(1, 1))
        np_list.append(jnp.sum(posf).reshape(1, 1))

    mine8 = jnp.concatenate(mine_rows, axis=0)
    ce08 = jnp.concatenate(ce0_rows, axis=0)
    ce18 = jnp.concatenate(ce1_rows, axis=0)
    pos8 = jnp.concatenate(pos_rows, axis=0) > 0.0
    npos8 = jnp.concatenate(np_list, axis=0)
    k8 = jnp.minimum(_NEGPOS * npos8.astype(jnp.int32), num_p - 1)

    bits = jax.lax.bitcast_convert_type(mine8, jnp.int32)

    def _count_gt(t):
        return jnp.sum((bits > t).astype(jnp.int32), axis=1, keepdims=True)

    def _step(_, lohi):
        lo, hi = lohi
        mid = (lo + hi) // 2
        ge = _count_gt(mid) >= k8
        return jnp.where(ge, mid, lo), jnp.where(ge, hi, mid)

    lo0 = jnp.full((img, 1), -1, jnp.int32)
    hi0 = jnp.full((img, 1), _MAXFLOAT_BITS, jnp.int32)
    _, thr = jax.lax.fori_loop(0, 31, _step, (lo0, hi0))
    n_gt = _count_gt(thr)
    rem = k8 - n_gt
    eqm = bits == thr
    eqi = eqm.astype(jnp.int32)
    cum = eqi
    shift = 1
    while shift < num_p:
        cum = cum + jnp.concatenate(
            [jnp.zeros((img, shift), jnp.int32), cum[:, :num_p - shift]],
            axis=1)
        shift *= 2
    sel_neg = ((bits > thr) | (eqm & ((cum - eqi) < rem))) & (k8 > 0)

    lossc8 = jnp.sum(jnp.where(pos8, ce18, jnp.where(sel_neg, ce08, 0.0)),
                     axis=1, keepdims=True)
    lossl8 = jnp.concatenate(ll_list, axis=0)
    losslm8 = jnp.concatenate(lm_list, axis=0)

    zeros = jnp.zeros((img, 124), f32)
    out_ref[0, :, :] = jnp.concatenate(
        [lossl8, lossc8, losslm8, npos8, zeros], axis=1)


def kernel(loc_data, conf_data, landm_data, priors, targets):
    b, p = loc_data.shape[0], loc_data.shape[1]
    nobj = targets.shape[1]
    img = 8 if b % 8 == 0 else 1
    steps = b // img
    loc_cm = jnp.swapaxes(loc_data, 1, 2)
    conf_cm = jnp.swapaxes(conf_data, 1, 2)
    landm_cm = jnp.swapaxes(landm_data, 1, 2)
    pri_cm = jnp.transpose(priors)

    parts = pl.pallas_call(
        _body,
        grid=(steps,),
        in_specs=[
            pl.BlockSpec((img, 4, p), lambda i: (i, 0, 0)),
            pl.BlockSpec((img, 2, p), lambda i: (i, 0, 0)),
            pl.BlockSpec((img, 10, p), lambda i: (i, 0, 0)),
            pl.BlockSpec((4, p), lambda i: (0, 0)),
            pl.BlockSpec((img, nobj, 15), lambda i: (i, 0, 0)),
        ],
        out_specs=pl.BlockSpec((1, img, 128), lambda i: (i, 0, 0)),
        out_shape=jax.ShapeDtypeStruct((steps, img, 128), jnp.float32),
        compiler_params=pltpu.CompilerParams(
            dimension_semantics=("parallel",)),
    )(loc_cm, conf_cm, landm_cm, pri_cm, targets)

    s = jnp.sum(parts[:, :, :4], axis=(0, 1))
    n = jnp.maximum(s[3], 1.0)
    return jnp.stack([s[0] / n, s[1] / n, s[2] / n])
